# Initial kernel scaffold; baseline (speedup 1.0000x reference)
import jax
import jax.numpy as jnp
from jax.experimental import pallas as pl


def kernel(x, emb_table, W1, b1, W2, b2, Wc, bc):
    raise NotImplementedError("write your pallas kernel here")


# speedup vs baseline: 13.9995x; 13.9995x over previous
import functools

import jax
import jax.numpy as jnp
from jax import lax
from jax.experimental import pallas as pl
from jax.experimental.pallas import tpu as pltpu
from jax.experimental.pallas import tpu_sc as plsc

VOCAB = 100000
EMB = 64
HID = 256
TAGS = 5
B = 4096
L = 200

NC = 2
NS = 16
NW = NC * NS
NB = B // NW
HALF = L // 2
ROWS_PER_W = NB * 2
NLANE = EMB // 16


def _pool_sums_sc(x2d, table):
    mesh = plsc.VectorSubcoreMesh(core_axis_name="c", subcore_axis_name="s")

    @functools.partial(
        pl.kernel,
        mesh=mesh,
        out_type=jax.ShapeDtypeStruct((B, EMB), jnp.float32),
        compiler_params=pltpu.CompilerParams(use_tc_tiling_on_sc=False),
        scratch_types=[
            pltpu.VMEM((ROWS_PER_W, HALF), jnp.int32),
            pltpu.VMEM((HALF, EMB), jnp.float32),
            pltpu.VMEM((HALF, EMB), jnp.float32),
            pltpu.VMEM((HALF, EMB), jnp.float32),
            pltpu.VMEM((HALF, EMB), jnp.float32),
            pltpu.VMEM((NB, EMB), jnp.float32),
            pltpu.SemaphoreType.DMA,
            pltpu.SemaphoreType.DMA,
        ],
    )
    def k(x_hbm, tab_hbm, out_hbm, idx_v, a0, a1, b0, b1, pooled_v, sem_a, sem_b):
        wid = lax.axis_index("s") * NC + lax.axis_index("c")
        row0 = wid * ROWS_PER_W
        pltpu.sync_copy(x_hbm.at[pl.ds(row0, ROWS_PER_W)], idx_v)

        def fire(r, dst0, dst1, sem):
            pltpu.async_copy(tab_hbm.at[idx_v.at[r]], dst0, sem)
            pltpu.async_copy(tab_hbm.at[idx_v.at[r + 1]], dst1, sem)

        def drain(dst0, dst1, sem):
            pltpu.make_async_copy(tab_hbm.at[idx_v.at[0]], dst0, sem).wait()
            pltpu.make_async_copy(tab_hbm.at[idx_v.at[1]], dst1, sem).wait()

        def accum(i, dst0, dst1):
            def body(j, accs):
                return tuple(
                    accs[ci]
                    + dst0[j, pl.ds(16 * ci, 16)]
                    + dst1[j, pl.ds(16 * ci, 16)]
                    for ci in range(NLANE)
                )

            accs = tuple(jnp.zeros((16,), jnp.float32) for _ in range(NLANE))
            accs = lax.fori_loop(0, HALF, body, accs)
            for ci in range(NLANE):
                pooled_v[i, pl.ds(16 * ci, 16)] = accs[ci]

        fire(0, a0, a1, sem_a)

        def outer(kk, carry):
            i0 = 2 * kk
            i1 = i0 + 1
            fire(2 * i1, b0, b1, sem_b)
            drain(a0, a1, sem_a)
            accum(i0, a0, a1)

            @pl.when(kk < NB // 2 - 1)
            def _():
                fire(2 * (i1 + 1), a0, a1, sem_a)

            drain(b0, b1, sem_b)
            accum(i1, b0, b1)
            return carry

        lax.fori_loop(0, NB // 2, outer, 0)
        pltpu.sync_copy(pooled_v, out_hbm.at[pl.ds(wid * NB, NB)])

    return k(x2d, table)


def _mlp_tc(sums, W1, b1, W2, b2, Wcp, bcp):
    def body(s_ref, w1_ref, b1_ref, w2_ref, b2_ref, wc_ref, bc_ref, o_ref):
        p = s_ref[...] * (1.0 / L)
        h = jnp.dot(p, w1_ref[...], preferred_element_type=jnp.float32)
        h = jnp.maximum(h + b1_ref[...], 0.0)
        h = jnp.dot(h, w2_ref[...], preferred_element_type=jnp.float32)
        h = jnp.maximum(h + b2_ref[...], 0.0)
        o_ref[...] = (
            jnp.dot(h, wc_ref[...], preferred_element_type=jnp.float32)
            + bc_ref[...]
        )

    return pl.pallas_call(
        body,
        out_shape=jax.ShapeDtypeStruct((B, 128), jnp.float32),
    )(sums, W1, b1.reshape(1, HID), W2, b2.reshape(1, HID), Wcp,
      bcp.reshape(1, 128))


def kernel(x, emb_table, W1, b1, W2, b2, Wc, bc):
    x2d = x.astype(jnp.int32).reshape(B * L // HALF, HALF)
    sums = _pool_sums_sc(x2d, emb_table)
    Wcp = jnp.pad(Wc, ((0, 0), (0, 128 - TAGS)))
    bcp = jnp.pad(bc, (0, 128 - TAGS))
    out = _mlp_tc(sums, W1, b1, W2, b2, Wcp, bcp)
    return out[:, :TAGS]
